# baseline (speedup 1.0000x reference)
import functools

import jax
import jax.numpy as jnp
from jax.experimental import pallas as pl
from jax.experimental.pallas import tpu as pltpu

_REG = 1e-06
_LANES = 512


_LOG2E = 1.4426950408889634
_LN2 = 0.6931471805599453


def _combined(x, y, yd, cw):
    t = jnp.exp2(jnp.abs(x) * (-_LOG2E))
    softplus = _LN2 * jnp.log2(1.0 + t)
    bce = jnp.maximum(x, 0.0) - x * y + softplus
    lg2 = jnp.log2(yd)
    return bce + (cw * _LN2 * _LN2) * (lg2 * lg2)


_SLAB = 64


def _single_block_kernel(x_ref, y_ref, yd_ref, out_ref, *, cw, inv_n):
    v = _combined(x_ref[...], y_ref[...], yd_ref[...], cw)
    out_ref[...] = jnp.broadcast_to(jnp.sum(v) * inv_n, out_ref.shape)


def _stream_kernel(*refs, cw, inv_n, tile_rows, d, split):
    out_ref, acc = refs[-2], refs[-1]
    xs, ys, yds = refs[:split], refs[split:2 * split], refs[2 * split:3 * split]
    i = pl.program_id(1)

    @pl.when(i == 0)
    def _():
        acc[...] = jnp.zeros_like(acc)

    total = jnp.zeros((8, d), jnp.float32)
    for s in range(split):
        for g in range(tile_rows // _SLAB):
            sl = pl.ds(g * _SLAB, _SLAB)
            v = _combined(xs[s][sl, :], ys[s][sl, :], yds[s][sl, :], cw)
            total += v.reshape(_SLAB // 8, 8, d).sum(axis=0)
    acc[...] += total

    @pl.when(i == pl.num_programs(1) - 1)
    def _():
        out_ref[pl.program_id(0)] = jnp.sum(acc[...]) * inv_n


_BIG = 2048
_NSLOT = 2
_SMALLS = (1024, 512, 512)


def _chunk_sum(xbuf, ybuf, ydbuf, row0, nrows, cw, d):
    total = jnp.zeros((8, d), jnp.float32)
    for g in range(nrows // _SLAB):
        sl = pl.ds(row0 + g * _SLAB, _SLAB)
        v = _combined(xbuf[sl, :], ybuf[sl, :], ydbuf[sl, :], cw)
        total += v.reshape(_SLAB // 8, 8, d).sum(axis=0)
    return total


def _ring_kernel(x_hbm, y_hbm, yd_hbm, out_ref, xb, yb, ydb, xs, ys, yds,
                 sem_big, sem_small, *, cw, inv_n, n_bigs, d):
    hbms = (x_hbm, y_hbm, yd_hbm)
    rings = (xb, yb, ydb)
    smallbufs = (xs, ys, yds)

    def start_big(c, slot):
        for k in range(3):
            pltpu.make_async_copy(
                hbms[k].at[pl.ds(c * _BIG, _BIG), :],
                rings[k].at[slot],
                sem_big.at[k, slot],
            ).start()

    def wait_big(c, slot):
        for k in range(3):
            pltpu.make_async_copy(
                hbms[k].at[pl.ds(c * _BIG, _BIG), :],
                rings[k].at[slot],
                sem_big.at[k, slot],
            ).wait()

    small_offs = []
    off = 0
    for n in _SMALLS:
        small_offs.append(off)
        off += n

    def small_copy(j, k):
        row0 = n_bigs * _BIG + small_offs[j]
        return pltpu.make_async_copy(
            hbms[k].at[pl.ds(row0, _SMALLS[j]), :],
            smallbufs[k].at[pl.ds(small_offs[j], _SMALLS[j]), :],
            sem_small.at[k, j],
        )

    start_big(0, 0)

    def body(c, total):
        slot = jax.lax.rem(c, _NSLOT)
        wait_big(c, slot)
        start_big(c + 1, 1 - slot)
        total += _chunk_sum(xb.at[slot], yb.at[slot], ydb.at[slot],
                            0, _BIG, cw, d)
        return total

    total = jax.lax.fori_loop(0, n_bigs - 1, body,
                              jnp.zeros((8, d), jnp.float32))

    slot_last = (n_bigs - 1) % _NSLOT
    wait_big(n_bigs - 1, slot_last)
    for k in range(3):
        small_copy(0, k).start()
    total += _chunk_sum(xb.at[slot_last], yb.at[slot_last], ydb.at[slot_last],
                        0, _BIG, cw, d)

    for j, n in enumerate(_SMALLS):
        for k in range(3):
            small_copy(j, k).wait()
        if j + 1 < len(_SMALLS):
            for k in range(3):
                small_copy(j + 1, k).start()
        total += _chunk_sum(xs, ys, yds, small_offs[j], n, cw, d)

    out_ref[0] = jnp.sum(total) * inv_n


@functools.partial(jax.jit, static_argnames=("tile_rows", "split", "n_chunks"))
def _my_loss(x, y, y_d, tile_rows=1024, split=2, n_chunks=2):
    n_total = x.size
    inv_n = 1.0 / float(n_total)
    cw = _REG * float(n_total)

    d = _LANES
    if x.ndim >= 2 and x.shape[-1] == d and (x.size // d) % 8 == 0:
        x2 = x.reshape(-1, d)
        y2 = y.reshape(-1, d)
        yd2 = y_d.reshape(-1, d)
    else:
        pad = (-n_total) % (8 * d)

        def prep(a, pad_val):
            a = a.reshape(-1)
            if pad:
                a = jnp.pad(a, (0, pad), constant_values=pad_val)
            return a.reshape(-1, d)

        x2, y2, yd2 = prep(x, -1e4), prep(y, 0.0), prep(y_d, 1.0)

    n_rows = x2.shape[0]

    if n_rows <= 1024:
        out = pl.pallas_call(
            functools.partial(_single_block_kernel, cw=cw, inv_n=inv_n),
            out_shape=jax.ShapeDtypeStruct((8, 128), jnp.float32),
            compiler_params=pltpu.CompilerParams(
                vmem_limit_bytes=48 << 20),
        )(x2, y2, yd2)
        return out[0, 0]

    cost = pl.CostEstimate(
        flops=12 * n_rows * d,
        transcendentals=3 * n_rows * d,
        bytes_accessed=3 * n_rows * d * 4 + 512,
    )

    if n_rows % _BIG == 0 and n_rows // _BIG >= 4:
        n_bigs = n_rows // _BIG - 1
        out = pl.pallas_call(
            functools.partial(_ring_kernel, cw=cw, inv_n=inv_n,
                              n_bigs=n_bigs, d=d),
            out_shape=jax.ShapeDtypeStruct((1,), jnp.float32),
            in_specs=[pl.BlockSpec(memory_space=pl.ANY)] * 3,
            out_specs=pl.BlockSpec(memory_space=pltpu.SMEM),
            scratch_shapes=[
                pltpu.VMEM((_NSLOT, _BIG, d), jnp.float32),
                pltpu.VMEM((_NSLOT, _BIG, d), jnp.float32),
                pltpu.VMEM((_NSLOT, _BIG, d), jnp.float32),
                pltpu.VMEM((_BIG, d), jnp.float32),
                pltpu.VMEM((_BIG, d), jnp.float32),
                pltpu.VMEM((_BIG, d), jnp.float32),
                pltpu.SemaphoreType.DMA((3, _NSLOT)),
                pltpu.SemaphoreType.DMA((3, len(_SMALLS))),
            ],
            compiler_params=pltpu.CompilerParams(
                vmem_limit_bytes=52 << 20),
            cost_estimate=cost,
        )(x2, y2, yd2)
        return out.reshape(())

    quantum = n_chunks * split * tile_rows
    row_pad = (-n_rows) % quantum
    if row_pad:
        x2 = jnp.pad(x2, ((0, row_pad), (0, 0)), constant_values=-1e4)
        y2 = jnp.pad(y2, ((0, row_pad), (0, 0)), constant_values=0.0)
        yd2 = jnp.pad(yd2, ((0, row_pad), (0, 0)), constant_values=1.0)
        n_rows += row_pad
    steps = n_rows // quantum
    blocks_per_core = split * steps

    def make_spec(s):
        return pl.BlockSpec(
            (tile_rows, d),
            lambda p, i, _s=s: (p * blocks_per_core + _s * steps + i, 0))

    specs = [make_spec(s) for s in range(split)]
    grid = (n_chunks, steps)

    tile_bytes = tile_rows * d * 4
    vmem_limit = int(min(2 * 3 * split * tile_bytes + (4 << 20), 52 << 20))

    out = pl.pallas_call(
        functools.partial(_stream_kernel, cw=cw, inv_n=inv_n,
                          tile_rows=tile_rows, d=d, split=split),
        out_shape=jax.ShapeDtypeStruct((n_chunks,), jnp.float32),
        grid=grid,
        in_specs=specs + specs + specs,
        out_specs=pl.BlockSpec(memory_space=pltpu.SMEM),
        scratch_shapes=[pltpu.VMEM((8, d), jnp.float32)],
        compiler_params=pltpu.CompilerParams(
            dimension_semantics=("parallel", "arbitrary"),
            vmem_limit_bytes=vmem_limit,
        ),
        cost_estimate=cost,
    )(*([x2] * split + [y2] * split + [yd2] * split))

    if n_chunks == 1:
        return out.reshape(())
    return jnp.sum(out)


def kernel(x, y, y_d):
    return _my_loss(x, y, y_d, tile_rows=2048, split=1, n_chunks=1)


# speedup vs baseline: 1.1871x; 1.1871x over previous
import functools

import jax
import jax.numpy as jnp
from jax.experimental import pallas as pl
from jax.experimental.pallas import tpu as pltpu

_REG = 1e-06
_LANES = 512


_LOG2E = 1.4426950408889634
_LN2 = 0.6931471805599453


def _combined(x, y, yd, cw):
    x = x.astype(jnp.float32)
    y = y.astype(jnp.float32)
    yd = yd.astype(jnp.float32)
    t = jnp.exp2(jnp.abs(x) * (-_LOG2E))
    softplus = _LN2 * jnp.log2(1.0 + t)
    bce = jnp.maximum(x, 0.0) - x * y + softplus
    lg2 = jnp.log2(yd)
    return bce + (cw * _LN2 * _LN2) * (lg2 * lg2)


_SLAB = 64


def _single_block_kernel(x_ref, y_ref, yd_ref, out_ref, *, cw, inv_n):
    v = _combined(x_ref[...], y_ref[...], yd_ref[...], cw)
    out_ref[...] = jnp.broadcast_to(jnp.sum(v) * inv_n, out_ref.shape)


def _stream_kernel(*refs, cw, inv_n, tile_rows, d, split):
    out_ref, acc = refs[-2], refs[-1]
    xs, ys, yds = refs[:split], refs[split:2 * split], refs[2 * split:3 * split]
    i = pl.program_id(1)

    @pl.when(i == 0)
    def _():
        acc[...] = jnp.zeros_like(acc)

    total = jnp.zeros((8, d), jnp.float32)
    for s in range(split):
        for g in range(tile_rows // _SLAB):
            sl = pl.ds(g * _SLAB, _SLAB)
            v = _combined(xs[s][sl, :], ys[s][sl, :], yds[s][sl, :], cw)
            total += v.reshape(_SLAB // 8, 8, d).sum(axis=0)
    acc[...] += total

    @pl.when(i == pl.num_programs(1) - 1)
    def _():
        out_ref[pl.program_id(0)] = jnp.sum(acc[...]) * inv_n


@functools.partial(jax.jit, static_argnames=("tile_rows", "split", "n_chunks"))
def _my_loss(x, y, y_d, tile_rows=1024, split=2, n_chunks=2):
    n_total = x.size
    inv_n = 1.0 / float(n_total)
    cw = _REG * float(n_total)

    d = _LANES
    if x.ndim >= 2 and x.shape[-1] == d and (x.size // d) % 8 == 0:
        x2 = x.reshape(-1, d)
        y2 = y.reshape(-1, d)
        yd2 = y_d.reshape(-1, d)
    else:
        pad = (-n_total) % (8 * d)

        def prep(a, pad_val):
            a = a.reshape(-1)
            if pad:
                a = jnp.pad(a, (0, pad), constant_values=pad_val)
            return a.reshape(-1, d)

        x2, y2, yd2 = prep(x, -1e4), prep(y, 0.0), prep(y_d, 1.0)

    n_rows = x2.shape[0]

    if n_rows <= 1024:
        out = pl.pallas_call(
            functools.partial(_single_block_kernel, cw=cw, inv_n=inv_n),
            out_shape=jax.ShapeDtypeStruct((8, 128), jnp.float32),
            compiler_params=pltpu.CompilerParams(
                vmem_limit_bytes=48 << 20),
        )(x2, y2, yd2)
        return out[0, 0]

    cost = pl.CostEstimate(
        flops=12 * n_rows * d,
        transcendentals=3 * n_rows * d,
        bytes_accessed=3 * n_rows * d * 4 + 512,
    )

    quantum = n_chunks * split * tile_rows
    row_pad = (-n_rows) % quantum
    if row_pad:
        x2 = jnp.pad(x2, ((0, row_pad), (0, 0)), constant_values=-1e4)
        y2 = jnp.pad(y2, ((0, row_pad), (0, 0)), constant_values=0.0)
        yd2 = jnp.pad(yd2, ((0, row_pad), (0, 0)), constant_values=1.0)
        n_rows += row_pad
    steps = n_rows // quantum
    blocks_per_core = split * steps

    def make_spec(s):
        return pl.BlockSpec(
            (tile_rows, d),
            lambda p, i, _s=s: (p * blocks_per_core + _s * steps + i, 0))

    specs = [make_spec(s) for s in range(split)]
    grid = (n_chunks, steps)

    tile_bytes = tile_rows * d * 4
    vmem_limit = int(min(2 * 3 * split * tile_bytes + (4 << 20), 52 << 20))

    out = pl.pallas_call(
        functools.partial(_stream_kernel, cw=cw, inv_n=inv_n,
                          tile_rows=tile_rows, d=d, split=split),
        out_shape=jax.ShapeDtypeStruct((n_chunks,), jnp.float32),
        grid=grid,
        in_specs=specs + specs + specs,
        out_specs=pl.BlockSpec(memory_space=pltpu.SMEM),
        scratch_shapes=[pltpu.VMEM((8, d), jnp.float32)],
        compiler_params=pltpu.CompilerParams(
            dimension_semantics=("parallel", "arbitrary"),
            vmem_limit_bytes=vmem_limit,
        ),
        cost_estimate=cost,
    )(*([x2] * split + [y2] * split + [yd2] * split))

    if n_chunks == 1:
        return out.reshape(())
    return jnp.sum(out)


def kernel(x, y, y_d):
    return _my_loss(x, y, y_d, tile_rows=2048, split=1, n_chunks=1)
